# baseline (device time: 123584 ns/iter reference)
import functools

import jax
import jax.numpy as jnp
from jax import lax
from jax.experimental import pallas as pl
from jax.experimental.pallas import tpu as pltpu

N_DEV = 16
SQ = 2048
D_MODEL = 1024
H_LOC = 8
DH = 128
N_RES = 4
N_GRP = 8
ROWS = 64
QR = 512
CH = 128
HALF = D_MODEL // 2
SCALE = 0.08838834764831843


def _body(x_ref, wq_ref, k_hbm, v_hbm, wo_ref, out_ref,
          k_vmem, v_vmem, ctx_vmem,
          rsa_stage_a, rsa_buf_a, rsa_stage_b, rsa_buf_b,
          rsb_stage_a, rsb_buf_a, rsb_stage_b, rsb_buf_b,
          agb_stage_a, agb_buf_a, agb_stage_b, agb_buf_b,
          aga_stage_a, aga_buf_a, aga_stage_b, aga_buf_b,
          k_sem, v_sem,
          rsa_ss_a, rsa_rs_a, rsa_ss_b, rsa_rs_b,
          rsb_ss_a, rsb_rs_a, rsb_ss_b, rsb_rs_b,
          agb_ss_a, agb_rs_a, agb_ss_b, agb_rs_b,
          aga_ss_a, aga_rs_a, aga_ss_b, aga_rs_b):
    d = lax.axis_index("i")
    p = d % 4
    z = d // 4
    right_p = z * 4 + (p + 1) % 4
    left_p = z * 4 + (p - 1) % 4
    up = ((z + 1) % 4) * 4 + p
    down = ((z - 1) % 4) * 4 + p

    kcp = pltpu.make_async_copy(
        k_hbm.at[0, :, pl.ds(d * H_LOC, H_LOC), :], k_vmem, k_sem)
    vcp = pltpu.make_async_copy(
        v_hbm.at[0, :, pl.ds(d * H_LOC, H_LOC), :], v_vmem, v_sem)
    kcp.start()
    vcp.start()

    barrier_sem = pltpu.get_barrier_semaphore()
    for nbr in (left_p, right_p, up, down):
        pl.semaphore_signal(barrier_sem, inc=1, device_id=(nbr,),
                            device_id_type=pl.DeviceIdType.MESH)
    pl.semaphore_wait(barrier_sem, 4)

    bf16 = jnp.bfloat16
    f32 = jnp.float32
    wq = wq_ref[:, :]
    wo = wo_ref[:, :]
    x_all = x_ref[:, :].reshape(N_GRP, N_RES, ROWS, D_MODEL)

    def class_rows(ref, r):
        return jnp.concatenate(
            [ref[pl.ds((g * N_RES + r) * ROWS, ROWS)] for g in range(N_GRP)],
            axis=0)

    for r in range(N_RES):
        xr = x_all[:, r].reshape(N_GRP * ROWS, D_MODEL)
        qr = jnp.dot(xr, wq, preferred_element_type=f32).astype(bf16)
        if r == 0:
            kcp.wait()
            vcp.wait()
        kr = class_rows(k_vmem, r).astype(bf16).reshape(
            N_GRP * ROWS, H_LOC * DH)
        vr = class_rows(v_vmem, r).astype(bf16).reshape(
            N_GRP * ROWS, H_LOC * DH)
        ctx_parts = []
        for h in range(H_LOC):
            cols = slice(h * DH, (h + 1) * DH)
            e = jnp.exp(lax.dot_general(
                qr[:, cols], kr[:, cols], (((1,), (1,)), ((), ())),
                preferred_element_type=f32))
            denom = jnp.sum(e, axis=1, keepdims=True)
            ctx_h = jnp.dot(e.astype(bf16), vr[:, cols],
                            preferred_element_type=f32)
            ctx_parts.append(ctx_h / denom)
        ctx = jnp.concatenate(ctx_parts, axis=1).astype(bf16)
        cg = ctx.reshape(N_GRP, ROWS, H_LOC * DH)
        for g in range(N_GRP):
            ctx_vmem[pl.ds((g * N_RES + r) * ROWS, ROWS), :] = cg[g]

    def pr_quarter(q):
        ctxq = ctx_vmem[pl.ds(q * QR, QR), :]
        out_ref[pl.ds(q * QR, QR), :] = jnp.dot(
            ctxq, wo, preferred_element_type=f32)

    def xchg(stage_a, buf_a, ss_a, rs_a, dev_a, src_rows_a, n_rows,
             stage_b, buf_b, ss_b, rs_b, dev_b, src_rows_b, step, stage=True):
        if stage:
            stage_a[:, :] = out_ref[pl.ds(src_rows_a, n_rows), :HALF].astype(bf16)
            stage_b[:, :] = out_ref[pl.ds(src_rows_b, n_rows), HALF:].astype(bf16)
            src_a, src_b = stage_a, stage_b
        else:
            src_a, src_b = buf_a.at[step - 1], buf_b.at[step - 1]
        ra = pltpu.make_async_remote_copy(
            src_ref=src_a, dst_ref=buf_a.at[step],
            send_sem=ss_a, recv_sem=rs_a.at[step],
            device_id=(dev_a,), device_id_type=pl.DeviceIdType.MESH,
        )
        rb = pltpu.make_async_remote_copy(
            src_ref=src_b, dst_ref=buf_b.at[step],
            send_sem=ss_b, recv_sem=rs_b.at[step],
            device_id=(dev_b,), device_id_type=pl.DeviceIdType.MESH,
        )
        ra.start()
        rb.start()
        return ra, rb

    pr_quarter(p)
    in_flight_work = {0: (-1, 1), 1: (2,), 2: ()}
    for s_ in range(3):
        ra, rb = xchg(
            rsa_stage_a, rsa_buf_a, rsa_ss_a, rsa_rs_a, right_p,
            ((p - s_) % 4) * QR, QR,
            rsa_stage_b, rsa_buf_b, rsa_ss_b, rsa_rs_b, left_p,
            ((p + s_) % 4) * QR, s_)
        for off in in_flight_work[s_]:
            pr_quarter((p + off) % 4)
        qa_r = ((p - s_ - 1) % 4) * QR
        qb_r = ((p + s_ + 1) % 4) * QR
        ra.wait()
        out_ref[pl.ds(qa_r, QR), :HALF] = (
            out_ref[pl.ds(qa_r, QR), :HALF] + rsa_buf_a[s_].astype(f32))
        rb.wait()
        out_ref[pl.ds(qb_r, QR), HALF:] = (
            out_ref[pl.ds(qb_r, QR), HALF:] + rsa_buf_b[s_].astype(f32))

    qa = (p + 1) % 4
    qb = (p - 1) % 4

    for s_ in range(3):
        ra, rb = xchg(
            rsb_stage_a, rsb_buf_a, rsb_ss_a, rsb_rs_a, up,
            qa * QR + ((z - s_) % 4) * CH, CH,
            rsb_stage_b, rsb_buf_b, rsb_ss_b, rsb_rs_b, down,
            qb * QR + ((z + s_) % 4) * CH, s_)
        ua_r = qa * QR + ((z - s_ - 1) % 4) * CH
        ub_r = qb * QR + ((z + s_ + 1) % 4) * CH
        ra.wait()
        out_ref[pl.ds(ua_r, CH), :HALF] = (
            out_ref[pl.ds(ua_r, CH), :HALF] + rsb_buf_a[s_].astype(f32))
        rb.wait()
        out_ref[pl.ds(ub_r, CH), HALF:] = (
            out_ref[pl.ds(ub_r, CH), HALF:] + rsb_buf_b[s_].astype(f32))

    def agb_store(t):
        out_ref[pl.ds(qa * QR + ((z - t) % 4) * CH, CH), :HALF] = (
            agb_buf_a[t].astype(f32))
        out_ref[pl.ds(qb * QR + ((z + t) % 4) * CH, CH), HALF:] = (
            agb_buf_b[t].astype(f32))

    for t in range(3):
        ra, rb = xchg(
            agb_stage_a, agb_buf_a, agb_ss_a, agb_rs_a, up,
            qa * QR + ((z + 1) % 4) * CH, CH,
            agb_stage_b, agb_buf_b, agb_ss_b, agb_rs_b, down,
            qb * QR + ((z - 1) % 4) * CH, t, stage=(t == 0))
        if t > 0:
            agb_store(t - 1)
        ra.wait()
        rb.wait()
    agb_store(2)

    def aga_store(t):
        out_ref[pl.ds(((p - t) % 4) * QR, QR), :HALF] = (
            aga_buf_a[t].astype(f32))
        out_ref[pl.ds(((p + t) % 4) * QR, QR), HALF:] = (
            aga_buf_b[t].astype(f32))

    for t in range(3):
        ra, rb = xchg(
            aga_stage_a, aga_buf_a, aga_ss_a, aga_rs_a, right_p,
            qa * QR, QR,
            aga_stage_b, aga_buf_b, aga_ss_b, aga_rs_b, left_p,
            qb * QR, t, stage=(t == 0))
        if t > 0:
            aga_store(t - 1)
        ra.wait()
        rb.wait()
    aga_store(2)

    @functools.partial(pl.run_scoped, exit_sem=pltpu.SemaphoreType.REGULAR)
    def _(exit_sem):
        for nbr in (left_p, right_p, up, down):
            pl.semaphore_signal(exit_sem, inc=1, device_id=(nbr,),
                                device_id_type=pl.DeviceIdType.MESH)
        pl.semaphore_wait(exit_sem, 4)


def kernel(x, Wq, K_ext, V_ext, Wo):
    bf = jnp.bfloat16
    out = pl.pallas_call(
        _body,
        out_shape=jax.ShapeDtypeStruct((SQ, D_MODEL), jnp.float32),
        in_specs=[
            pl.BlockSpec(memory_space=pltpu.VMEM),
            pl.BlockSpec(memory_space=pltpu.VMEM),
            pl.BlockSpec(memory_space=pl.ANY),
            pl.BlockSpec(memory_space=pl.ANY),
            pl.BlockSpec(memory_space=pltpu.VMEM),
        ],
        out_specs=pl.BlockSpec(memory_space=pltpu.VMEM),
        scratch_shapes=[
            pltpu.VMEM((SQ, H_LOC, DH), jnp.float32),
            pltpu.VMEM((SQ, H_LOC, DH), jnp.float32),
            pltpu.VMEM((SQ, H_LOC * DH), bf),
            pltpu.VMEM((QR, HALF), bf), pltpu.VMEM((3, QR, HALF), bf),
            pltpu.VMEM((QR, HALF), bf), pltpu.VMEM((3, QR, HALF), bf),
            pltpu.VMEM((CH, HALF), bf), pltpu.VMEM((3, CH, HALF), bf),
            pltpu.VMEM((CH, HALF), bf), pltpu.VMEM((3, CH, HALF), bf),
            pltpu.VMEM((CH, HALF), bf), pltpu.VMEM((3, CH, HALF), bf),
            pltpu.VMEM((CH, HALF), bf), pltpu.VMEM((3, CH, HALF), bf),
            pltpu.VMEM((QR, HALF), bf), pltpu.VMEM((3, QR, HALF), bf),
            pltpu.VMEM((QR, HALF), bf), pltpu.VMEM((3, QR, HALF), bf),
            pltpu.SemaphoreType.DMA,
            pltpu.SemaphoreType.DMA,
        ] + [
            pltpu.SemaphoreType.DMA, pltpu.SemaphoreType.DMA((3,)),
            pltpu.SemaphoreType.DMA, pltpu.SemaphoreType.DMA((3,)),
        ] * 4,
        compiler_params=pltpu.CompilerParams(
            collective_id=0, vmem_limit_bytes=100 * 1024 * 1024
        ),
    )(
        x[0].astype(jnp.bfloat16),
        (Wq * SCALE).astype(jnp.bfloat16),
        K_ext,
        V_ext,
        Wo.astype(jnp.bfloat16),
    )
    return out[None]


# device time: 116185 ns/iter; 1.0637x vs baseline; 1.0637x over previous
import functools

import jax
import jax.numpy as jnp
from jax import lax
from jax.experimental import pallas as pl
from jax.experimental.pallas import tpu as pltpu

N_DEV = 16
SQ = 2048
D_MODEL = 1024
H_LOC = 8
DH = 128
N_RES = 4
N_GRP = 8
ROWS = 64
QR = 512
CH = 128
HALF = D_MODEL // 2
SCALE = 0.08838834764831843


def _body(x_ref, wq_ref, k_hbm, v_hbm, wo_ref, out_ref,
          k_vmem, v_vmem, ctx_vmem,
          rsa_stage_a, rsa_buf_a, rsa_stage_b, rsa_buf_b,
          rsb_stage_a, rsb_buf_a, rsb_stage_b, rsb_buf_b,
          k_sem, v_sem,
          rsa_ss_a, rsa_rs_a, rsa_ss_b, rsa_rs_b,
          rsb_ss_a, rsb_rs_a, rsb_ss_b, rsb_rs_b,
          agb_ss_a, agb_rs_a, agb_ss_b, agb_rs_b,
          aga_ss_a, aga_rs_a, aga_ss_b, aga_rs_b):
    d = lax.axis_index("i")
    p = d % 4
    z = d // 4
    right_p = z * 4 + (p + 1) % 4
    left_p = z * 4 + (p - 1) % 4
    up = ((z + 1) % 4) * 4 + p
    down = ((z - 1) % 4) * 4 + p

    kcp = pltpu.make_async_copy(
        k_hbm.at[0, :, pl.ds(d * H_LOC, H_LOC), :], k_vmem, k_sem)
    vcp = pltpu.make_async_copy(
        v_hbm.at[0, :, pl.ds(d * H_LOC, H_LOC), :], v_vmem, v_sem)
    kcp.start()
    vcp.start()

    barrier_sem = pltpu.get_barrier_semaphore()
    for nbr in (left_p, right_p, up, down):
        pl.semaphore_signal(barrier_sem, inc=1, device_id=(nbr,),
                            device_id_type=pl.DeviceIdType.MESH)
    pl.semaphore_wait(barrier_sem, 4)

    bf16 = jnp.bfloat16
    f32 = jnp.float32
    wq = (wq_ref[:, :] * SCALE).astype(bf16)
    wo = wo_ref[:, :].astype(bf16)
    x_all = x_ref[:, :].astype(bf16).reshape(N_GRP, N_RES, ROWS, D_MODEL)

    k_all = v_all = None
    for r in range(N_RES):
        xr = x_all[:, r].reshape(N_GRP * ROWS, D_MODEL)
        qr = jnp.dot(xr, wq, preferred_element_type=f32).astype(bf16)
        if r == 0:
            kcp.wait()
            vcp.wait()
            k_all = k_vmem[:, :, :].astype(bf16).reshape(
                N_GRP, N_RES, ROWS, H_LOC * DH)
            v_all = v_vmem[:, :, :].astype(bf16).reshape(
                N_GRP, N_RES, ROWS, H_LOC * DH)
        kr = k_all[:, r].reshape(N_GRP * ROWS, H_LOC * DH)
        vr = v_all[:, r].reshape(N_GRP * ROWS, H_LOC * DH)
        ctx_parts = []
        for h in range(H_LOC):
            cols = slice(h * DH, (h + 1) * DH)
            e = jnp.exp(lax.dot_general(
                qr[:, cols], kr[:, cols], (((1,), (1,)), ((), ())),
                preferred_element_type=f32))
            denom = jnp.sum(e, axis=1, keepdims=True)
            ctx_h = jnp.dot(e.astype(bf16), vr[:, cols],
                            preferred_element_type=f32)
            ctx_parts.append(ctx_h / denom)
        ctx = jnp.concatenate(ctx_parts, axis=1).astype(bf16)
        cg = ctx.reshape(N_GRP, ROWS, H_LOC * DH)
        for g in range(N_GRP):
            ctx_vmem[pl.ds((g * N_RES + r) * ROWS, ROWS), :] = cg[g]

    def pr_quarter(q):
        ctxq = ctx_vmem[pl.ds(q * QR, QR), :]
        out_ref[pl.ds(q * QR, QR), :] = jnp.dot(
            ctxq, wo, preferred_element_type=f32)

    def xchg(stage_a, buf_a, ss_a, rs_a, dev_a, src_rows_a, n_rows,
             stage_b, buf_b, ss_b, rs_b, dev_b, src_rows_b, step, stage=True):
        if stage:
            stage_a[:, :] = out_ref[pl.ds(src_rows_a, n_rows), :HALF].astype(bf16)
            stage_b[:, :] = out_ref[pl.ds(src_rows_b, n_rows), HALF:].astype(bf16)
            src_a, src_b = stage_a, stage_b
        else:
            src_a, src_b = buf_a.at[step - 1], buf_b.at[step - 1]
        ra = pltpu.make_async_remote_copy(
            src_ref=src_a, dst_ref=buf_a.at[step],
            send_sem=ss_a, recv_sem=rs_a.at[step],
            device_id=(dev_a,), device_id_type=pl.DeviceIdType.MESH,
        )
        rb = pltpu.make_async_remote_copy(
            src_ref=src_b, dst_ref=buf_b.at[step],
            send_sem=ss_b, recv_sem=rs_b.at[step],
            device_id=(dev_b,), device_id_type=pl.DeviceIdType.MESH,
        )
        ra.start()
        rb.start()
        return ra, rb

    pr_quarter(p)
    in_flight_work = {0: (-1, 1), 1: (2,), 2: ()}
    for s_ in range(3):
        ra, rb = xchg(
            rsa_stage_a, rsa_buf_a, rsa_ss_a, rsa_rs_a, right_p,
            ((p - s_) % 4) * QR, QR,
            rsa_stage_b, rsa_buf_b, rsa_ss_b, rsa_rs_b, left_p,
            ((p + s_) % 4) * QR, s_)
        for off in in_flight_work[s_]:
            pr_quarter((p + off) % 4)
        qa_r = ((p - s_ - 1) % 4) * QR
        qb_r = ((p + s_ + 1) % 4) * QR
        ra.wait()
        out_ref[pl.ds(qa_r, QR), :HALF] = (
            out_ref[pl.ds(qa_r, QR), :HALF] + rsa_buf_a[s_].astype(f32))
        rb.wait()
        out_ref[pl.ds(qb_r, QR), HALF:] = (
            out_ref[pl.ds(qb_r, QR), HALF:] + rsa_buf_b[s_].astype(f32))

    qa = (p + 1) % 4
    qb = (p - 1) % 4

    for s_ in range(3):
        ra, rb = xchg(
            rsb_stage_a, rsb_buf_a, rsb_ss_a, rsb_rs_a, up,
            qa * QR + ((z - s_) % 4) * CH, CH,
            rsb_stage_b, rsb_buf_b, rsb_ss_b, rsb_rs_b, down,
            qb * QR + ((z + s_) % 4) * CH, s_)
        ua_r = qa * QR + ((z - s_ - 1) % 4) * CH
        ub_r = qb * QR + ((z + s_ + 1) % 4) * CH
        ra.wait()
        out_ref[pl.ds(ua_r, CH), :HALF] = (
            out_ref[pl.ds(ua_r, CH), :HALF] + rsb_buf_a[s_].astype(f32))
        rb.wait()
        out_ref[pl.ds(ub_r, CH), HALF:] = (
            out_ref[pl.ds(ub_r, CH), HALF:] + rsb_buf_b[s_].astype(f32))

    def agb_store(t):
        out_ref[pl.ds(qa * QR + ((z - t) % 4) * CH, CH), :HALF] = (
            rsb_buf_a[t].astype(f32))
        out_ref[pl.ds(qb * QR + ((z + t) % 4) * CH, CH), HALF:] = (
            rsb_buf_b[t].astype(f32))

    for t in range(3):
        ra, rb = xchg(
            rsb_stage_a, rsb_buf_a, agb_ss_a, agb_rs_a, up,
            qa * QR + ((z + 1) % 4) * CH, CH,
            rsb_stage_b, rsb_buf_b, agb_ss_b, agb_rs_b, down,
            qb * QR + ((z - 1) % 4) * CH, t, stage=(t == 0))
        if t > 0:
            agb_store(t - 1)
        ra.wait()
        rb.wait()
    agb_store(2)

    def aga_store(t):
        out_ref[pl.ds(((p - t) % 4) * QR, QR), :HALF] = (
            rsa_buf_a[t].astype(f32))
        out_ref[pl.ds(((p + t) % 4) * QR, QR), HALF:] = (
            rsa_buf_b[t].astype(f32))

    for t in range(3):
        ra, rb = xchg(
            rsa_stage_a, rsa_buf_a, aga_ss_a, aga_rs_a, right_p,
            qa * QR, QR,
            rsa_stage_b, rsa_buf_b, aga_ss_b, aga_rs_b, left_p,
            qb * QR, t, stage=(t == 0))
        if t > 0:
            aga_store(t - 1)
        ra.wait()
        rb.wait()
    aga_store(2)

    @functools.partial(pl.run_scoped, exit_sem=pltpu.SemaphoreType.REGULAR)
    def _(exit_sem):
        for nbr in (left_p, right_p, up, down):
            pl.semaphore_signal(exit_sem, inc=1, device_id=(nbr,),
                                device_id_type=pl.DeviceIdType.MESH)
        pl.semaphore_wait(exit_sem, 4)


def kernel(x, Wq, K_ext, V_ext, Wo):
    bf = jnp.bfloat16
    out = pl.pallas_call(
        _body,
        out_shape=jax.ShapeDtypeStruct((SQ, D_MODEL), jnp.float32),
        in_specs=[
            pl.BlockSpec(memory_space=pltpu.VMEM),
            pl.BlockSpec(memory_space=pltpu.VMEM),
            pl.BlockSpec(memory_space=pl.ANY),
            pl.BlockSpec(memory_space=pl.ANY),
            pl.BlockSpec(memory_space=pltpu.VMEM),
        ],
        out_specs=pl.BlockSpec(memory_space=pltpu.VMEM),
        scratch_shapes=[
            pltpu.VMEM((SQ, H_LOC, DH), jnp.float32),
            pltpu.VMEM((SQ, H_LOC, DH), jnp.float32),
            pltpu.VMEM((SQ, H_LOC * DH), bf),
            pltpu.VMEM((QR, HALF), bf), pltpu.VMEM((3, QR, HALF), bf),
            pltpu.VMEM((QR, HALF), bf), pltpu.VMEM((3, QR, HALF), bf),
            pltpu.VMEM((CH, HALF), bf), pltpu.VMEM((3, CH, HALF), bf),
            pltpu.VMEM((CH, HALF), bf), pltpu.VMEM((3, CH, HALF), bf),
            pltpu.SemaphoreType.DMA,
            pltpu.SemaphoreType.DMA,
        ] + [
            pltpu.SemaphoreType.DMA, pltpu.SemaphoreType.DMA((3,)),
            pltpu.SemaphoreType.DMA, pltpu.SemaphoreType.DMA((3,)),
        ] * 4,
        compiler_params=pltpu.CompilerParams(
            collective_id=0, vmem_limit_bytes=100 * 1024 * 1024
        ),
    )(x[0], Wq, K_ext, V_ext, Wo)
    return out[None]
